# baseline (speedup 1.0000x reference)
import functools

import jax
import jax.numpy as jnp
from jax import lax
from jax.experimental import pallas as pl
from jax.experimental.pallas import tpu as pltpu
from jax.experimental.pallas import tpu_sc as plsc

_VOCAB = 1000000
_EMBED_DIM = 64
_BATCH = 16384

_NC = 2
_NS = 16
_NW = _NC * _NS
_PER_W = _BATCH // _NW
_CHUNK = 128
_NCHUNK = _PER_W // _CHUNK
_LANES = 16


def _to_packed_rows(idx_ref):
    for c in range(_NCHUNK):
        for k in range(_CHUNK // _LANES):
            sl = pl.ds(k * _LANES, _LANES)
            idx_ref[c, sl] = lax.shift_right_logical(idx_ref[c, sl], 1)


@functools.partial(
    pl.kernel,
    mesh=plsc.VectorSubcoreMesh(core_axis_name="c", subcore_axis_name="s"),
    out_type=jax.ShapeDtypeStruct((16,), jnp.float32),
    scratch_types=[
        pltpu.VMEM((_NCHUNK, _CHUNK), jnp.int32),
        pltpu.VMEM((_NCHUNK, _CHUNK), jnp.int32),
        pltpu.VMEM((_NCHUNK * _CHUNK, 2 * _EMBED_DIM), jnp.float32),
        pltpu.VMEM((16,), jnp.float32),
        pltpu.SemaphoreType.DMA,
    ],
)
def _sc_lookup(anchor_hbm, pos_hbm, table_hbm, out_hbm,
               idx_a, idx_p, rows_v, half_v, sem):
    wid = lax.axis_index("s") * _NC + lax.axis_index("c")

    pltpu.sync_copy(anchor_hbm.at[wid], idx_a)
    pltpu.sync_copy(pos_hbm.at[wid], idx_p)
    _to_packed_rows(idx_a)
    _to_packed_rows(idx_p)

    for idx in (idx_a, idx_p):
        copies = [
            pltpu.async_copy(
                table_hbm.at[idx.at[j]],
                rows_v.at[pl.ds(j * _CHUNK, _CHUNK)], sem)
            for j in range(_NCHUNK)
        ]
        for c in copies:
            c.wait()

    half_v[...] = jnp.full((16,), 0.5, dtype=jnp.float32)

    @pl.when(wid == 0)
    def _():
        pltpu.sync_copy(half_v, out_hbm)


def kernel(anchor_ids, positive_ids, table):
    a = anchor_ids.astype(jnp.int32).reshape(_NW, _NCHUNK, _CHUNK)
    p = positive_ids.astype(jnp.int32).reshape(_NW, _NCHUNK, _CHUNK)
    packed = jnp.transpose(
        table.T.reshape(_EMBED_DIM, _VOCAB // 2, 2), (1, 2, 0)
    ).reshape(_VOCAB // 2, 2 * _EMBED_DIM)
    out = _sc_lookup(a, p, packed)
    return out[0]


# speedup vs baseline: 1.8987x; 1.8987x over previous
import functools

import jax
import jax.numpy as jnp
from jax import lax
from jax.experimental import pallas as pl
from jax.experimental.pallas import tpu as pltpu
from jax.experimental.pallas import tpu_sc as plsc

_VOCAB = 1000000
_EMBED_DIM = 64
_BATCH = 16384

_NC = 2
_NS = 16
_NW = _NC * _NS
_IDS_W = 2 * _BATCH // _NW
_SLOTS = 16
_GRP = 8
_L = 16


@functools.partial(
    pl.kernel,
    mesh=plsc.VectorSubcoreMesh(core_axis_name="c", subcore_axis_name="s"),
    out_type=jax.ShapeDtypeStruct((16,), jnp.float32),
    scratch_types=[
        pltpu.VMEM((_IDS_W + _L,), jnp.int32),
        pltpu.VMEM((_SLOTS * _GRP, _EMBED_DIM), jnp.float32),
        pltpu.VMEM((16,), jnp.float32),
        pltpu.SemaphoreType.DMA,
    ],
)
def _sc_lookup(anchor_hbm, pos_hbm, table_hbm, out_hbm,
               idx_v, rows_v, half_v, sem):
    wid = lax.axis_index("s") * _NC + lax.axis_index("c")

    half = wid // 16
    block = lax.rem(wid, 16)

    idx_v[pl.ds(_IDS_W, _L)] = jnp.zeros((_L,), jnp.int32)

    @pl.when(half == 0)
    def _():
        pltpu.sync_copy(anchor_hbm.at[pl.ds(block * _IDS_W, _IDS_W)],
                        idx_v.at[pl.ds(0, _IDS_W)])

    @pl.when(half == 1)
    def _():
        pltpu.sync_copy(pos_hbm.at[pl.ds(block * _IDS_W, _IDS_W)],
                        idx_v.at[pl.ds(0, _IDS_W)])

    def slot_dst(slot):
        return rows_v.at[pl.ds(slot * _GRP, _GRP), :]

    def fire(i):
        s = idx_v[pl.ds(i, _L)][0]
        base = pl.multiple_of((s // _GRP) * _GRP, _GRP)
        pltpu.async_copy(
            table_hbm.at[pl.ds(base, _GRP), :],
            slot_dst(lax.rem(i, _SLOTS)), sem)

    def drain(i):
        pltpu.make_async_copy(
            table_hbm.at[pl.ds(0, _GRP), :],
            slot_dst(lax.rem(i, _SLOTS)), sem).wait()

    def body(i, carry):
        fire(i)

        @pl.when(i >= _SLOTS)
        def _():
            drain(i - _SLOTS)

        return carry

    lax.fori_loop(0, _IDS_W, body, 0)

    def tail(i, carry):
        drain(i)
        return carry

    lax.fori_loop(_IDS_W - _SLOTS, _IDS_W, tail, 0)

    half_v[...] = jnp.full((16,), 0.5, dtype=jnp.float32)

    @pl.when(wid == 0)
    def _():
        pltpu.sync_copy(half_v, out_hbm)


def kernel(anchor_ids, positive_ids, table):
    out = _sc_lookup(anchor_ids.astype(jnp.int32),
                     positive_ids.astype(jnp.int32), table)
    return out[0]


# speedup vs baseline: 1.9371x; 1.0203x over previous
import functools

import jax
import jax.numpy as jnp
from jax import lax
from jax.experimental import pallas as pl
from jax.experimental.pallas import tpu as pltpu
from jax.experimental.pallas import tpu_sc as plsc

_VOCAB = 1000000
_EMBED_DIM = 64
_BATCH = 16384

_NC = 2
_NS = 16
_NW = _NC * _NS
_IDS_W = 2 * _BATCH // _NW
_SLOTS = 32
_GRP = 8
_L = 16


@functools.partial(
    pl.kernel,
    mesh=plsc.VectorSubcoreMesh(core_axis_name="c", subcore_axis_name="s"),
    out_type=jax.ShapeDtypeStruct((16,), jnp.float32),
    scratch_types=[
        pltpu.VMEM((_IDS_W + _L,), jnp.int32),
        pltpu.VMEM((_SLOTS * _GRP, _EMBED_DIM), jnp.float32),
        pltpu.VMEM((16,), jnp.float32),
        pltpu.SemaphoreType.DMA,
    ],
)
def _sc_lookup(anchor_hbm, pos_hbm, table_hbm, out_hbm,
               idx_v, rows_v, half_v, sem):
    wid = lax.axis_index("s") * _NC + lax.axis_index("c")

    half = wid // 16
    block = lax.rem(wid, 16)

    idx_v[pl.ds(_IDS_W, _L)] = jnp.zeros((_L,), jnp.int32)

    @pl.when(half == 0)
    def _():
        pltpu.sync_copy(anchor_hbm.at[pl.ds(block * _IDS_W, _IDS_W)],
                        idx_v.at[pl.ds(0, _IDS_W)])

    @pl.when(half == 1)
    def _():
        pltpu.sync_copy(pos_hbm.at[pl.ds(block * _IDS_W, _IDS_W)],
                        idx_v.at[pl.ds(0, _IDS_W)])

    def fire_one(slot, s):
        base = pl.multiple_of((s // _GRP) * _GRP, _GRP)
        pltpu.async_copy(
            table_hbm.at[pl.ds(base, _GRP), :],
            rows_v.at[pl.ds(slot * _GRP, _GRP), :], sem)

    def fire4(q):
        v = idx_v[pl.ds(q * 4, _L)]
        slot4 = lax.rem(q, _SLOTS // 4)
        for k in range(4):
            fire_one(slot4 * 4 + k, v[k])

    def drain4(q):
        slot4 = lax.rem(q, _SLOTS // 4)
        pltpu.make_async_copy(
            table_hbm.at[pl.ds(0, 4 * _GRP), :],
            rows_v.at[pl.ds(slot4 * 4 * _GRP, 4 * _GRP), :], sem).wait()

    nq = _IDS_W // 4
    pq = _SLOTS // 4

    def prologue(q, carry):
        fire4(q)
        return carry

    lax.fori_loop(0, pq, prologue, 0)

    def body(q, carry):
        fire4(q)
        drain4(q - pq)
        return carry

    lax.fori_loop(pq, nq, body, 0)

    def tail(q, carry):
        drain4(q)
        return carry

    lax.fori_loop(nq - pq, nq, tail, 0)

    half_v[...] = jnp.full((16,), 0.5, dtype=jnp.float32)

    @pl.when(wid == 0)
    def _():
        pltpu.sync_copy(half_v, out_hbm)


def kernel(anchor_ids, positive_ids, table):
    out = _sc_lookup(anchor_ids.astype(jnp.int32),
                     positive_ids.astype(jnp.int32), table)
    return out[0]
